# Initial kernel scaffold; baseline (speedup 1.0000x reference)
import functools

import jax
import jax.numpy as jnp
from jax import lax
from jax.experimental import pallas as pl
from jax.experimental.pallas import tpu as pltpu
from jax.experimental.pallas import tpu_sc as plsc

NC = 2
NS = 16
C = 128

_MESH = dict(core_axis_name="c", subcore_axis_name="s")


def _worker(cid, sid):
    return sid * NC + cid


def _sc_gather(table, idx, out_dim):
    e = idx.shape[0]
    n_chunks = e // C
    nloop = -(-n_chunks // (NC * NS))

    @functools.partial(
        pl.kernel,
        out_type=jax.ShapeDtypeStruct((e, out_dim), jnp.float32),
        mesh=plsc.VectorSubcoreMesh(**_MESH),
        scratch_types=[
            pltpu.VMEM((C,), jnp.int32),
            pltpu.VMEM((C, out_dim), jnp.float32),
            pltpu.SemaphoreType.DMA,
        ],
    )
    def k(table_hbm, idx_hbm, out_hbm, idx_v, rows_v, sem):
        wid = _worker(lax.axis_index("c"), lax.axis_index("s"))

        def body(i, carry):
            chunk = wid + i * (NC * NS)

            @pl.when(chunk < n_chunks)
            def _():
                base = chunk * C
                pltpu.sync_copy(idx_hbm.at[pl.ds(base, C)], idx_v)
                pltpu.async_copy(table_hbm.at[idx_v], rows_v, sem).wait()
                pltpu.sync_copy(rows_v, out_hbm.at[pl.ds(base, C)])

            return carry

        lax.fori_loop(0, nloop, body, 0)

    return k(table, idx)


def _sc_step(g, dst, src, qzero):
    e, d = g.shape
    n = qzero.shape[0]
    n_chunks = e // C
    scat_loop = -(-n_chunks // NS)
    half_chunks = (e // 2) // C
    gath_loop = -(-half_chunks // NS)
    rows_per_s = n // NS

    @functools.partial(
        pl.kernel,
        out_type=jax.ShapeDtypeStruct((e, d), jnp.float32),
        mesh=plsc.VectorSubcoreMesh(**_MESH),
        scratch_types=[
            pltpu.VMEM((C,), jnp.int32),
            pltpu.VMEM((C, d), jnp.float32),
            pltpu.VMEM_SHARED((n, d), jnp.float32),
            pltpu.SemaphoreType.DMA,
        ],
    )
    def k(g_hbm, dst_hbm, src_hbm, qz_hbm, out_hbm, idx_v, buf_v, q_sh, sem):
        cid = lax.axis_index("c")
        sid = lax.axis_index("s")

        pltpu.sync_copy(qz_hbm.at[pl.ds(sid * rows_per_s, rows_per_s)],
                        q_sh.at[pl.ds(sid * rows_per_s, rows_per_s)])
        plsc.subcore_barrier()

        def scat(i, carry):
            chunk = sid + i * NS

            @pl.when(chunk < n_chunks)
            def _():
                base = chunk * C
                pltpu.sync_copy(dst_hbm.at[pl.ds(base, C)], idx_v)
                pltpu.sync_copy(g_hbm.at[pl.ds(base, C)], buf_v)
                pltpu.sync_copy(buf_v, q_sh.at[idx_v], add=True)

            return carry

        lax.fori_loop(0, scat_loop, scat, 0)
        plsc.subcore_barrier()

        def gath(i, carry):
            chunk = sid + i * NS

            @pl.when(chunk < half_chunks)
            def _():
                base = cid * (e // 2) + chunk * C
                pltpu.sync_copy(src_hbm.at[pl.ds(base, C)], idx_v)
                pltpu.async_copy(q_sh.at[idx_v], buf_v, sem).wait()
                pltpu.sync_copy(buf_v, out_hbm.at[pl.ds(base, C)])

            return carry

        lax.fori_loop(0, gath_loop, gath, 0)

    return k(g, dst, src, qzero)


def _sc_segsum(rows, dst, qzero):
    e, d = rows.shape
    n = qzero.shape[0]
    n_chunks = e // C
    scat_loop = -(-n_chunks // NS)
    rows_per_s = n // NS
    out_rows = n // 2 // (NS // 2)

    @functools.partial(
        pl.kernel,
        out_type=jax.ShapeDtypeStruct((n, d), jnp.float32),
        mesh=plsc.VectorSubcoreMesh(**_MESH),
        scratch_types=[
            pltpu.VMEM((C,), jnp.int32),
            pltpu.VMEM((C, d), jnp.float32),
            pltpu.VMEM_SHARED((n, d), jnp.float32),
        ],
    )
    def k(rows_hbm, dst_hbm, qz_hbm, q_hbm, idx_v, buf_v, q_sh):
        cid = lax.axis_index("c")
        sid = lax.axis_index("s")

        pltpu.sync_copy(qz_hbm.at[pl.ds(sid * rows_per_s, rows_per_s)],
                        q_sh.at[pl.ds(sid * rows_per_s, rows_per_s)])
        plsc.subcore_barrier()

        def scat(i, carry):
            chunk = sid + i * NS

            @pl.when(chunk < n_chunks)
            def _():
                base = chunk * C
                pltpu.sync_copy(dst_hbm.at[pl.ds(base, C)], idx_v)
                pltpu.sync_copy(rows_hbm.at[pl.ds(base, C)], buf_v)
                pltpu.sync_copy(buf_v, q_sh.at[idx_v], add=True)

            return carry

        lax.fori_loop(0, scat_loop, scat, 0)
        plsc.subcore_barrier()

        @pl.when(sid < NS // 2)
        def _():
            base = cid * (n // 2) + sid * out_rows
            pltpu.sync_copy(q_sh.at[pl.ds(base, out_rows)],
                            q_hbm.at[pl.ds(base, out_rows)])

    return k(rows, dst, qzero)


def _tc_matmul(x, w, block_rows):
    m, kdim = x.shape
    _, nout = w.shape
    grid = m // block_rows

    def body(x_ref, w_ref, o_ref):
        o_ref[...] = jnp.dot(x_ref[...], w_ref[...],
                             preferred_element_type=jnp.float32)

    return pl.pallas_call(
        body,
        grid=(grid,),
        in_specs=[pl.BlockSpec((block_rows, kdim), lambda i: (i, 0)),
                  pl.BlockSpec((kdim, nout), lambda i: (0, 0))],
        out_specs=pl.BlockSpec((block_rows, nout), lambda i: (i, 0)),
        out_shape=jax.ShapeDtypeStruct((m, nout), jnp.float32),
    )(x, w)


def _tc_init(psrc, efeat, w1b, b_init, w_upd, block_rows):
    e, d = psrc.shape
    de = efeat.shape[1]
    grid = e // block_rows

    def body(p_ref, ef16_ref, w1b_ref, b_ref, wu_ref, ef_ref, g_ref):
        ef = p_ref[...] + jnp.dot(ef16_ref[...], w1b_ref[...],
                                  preferred_element_type=jnp.float32) + b_ref[...]
        ef = jnp.maximum(ef, 0.0)
        ef_ref[...] = ef
        g_ref[...] = jnp.dot(ef, wu_ref[...], preferred_element_type=jnp.float32)

    return pl.pallas_call(
        body,
        grid=(grid,),
        in_specs=[pl.BlockSpec((block_rows, d), lambda i: (i, 0)),
                  pl.BlockSpec((block_rows, de), lambda i: (i, 0)),
                  pl.BlockSpec((de, d), lambda i: (0, 0)),
                  pl.BlockSpec((1, d), lambda i: (0, 0)),
                  pl.BlockSpec((d, d), lambda i: (0, 0))],
        out_specs=[pl.BlockSpec((block_rows, d), lambda i: (i, 0)),
                   pl.BlockSpec((block_rows, d), lambda i: (i, 0))],
        out_shape=[jax.ShapeDtypeStruct((e, d), jnp.float32),
                   jax.ShapeDtypeStruct((e, d), jnp.float32)],
    )(psrc, efeat, w1b, b_init, w_upd)


def _tc_step(gathered, g, ef, b_upd, w_upd, block_rows, emit_g):
    e, d = g.shape
    grid = e // block_rows
    hb = grid // 2

    def body_g(ga_ref, gr_ref, ef_ref, b_ref, wu_ref, efo_ref, go_ref):
        efn = ga_ref[...] - gr_ref[...] + ef_ref[...] + b_ref[...]
        efn = jnp.maximum(efn, 0.0)
        efo_ref[...] = efn
        go_ref[...] = jnp.dot(efn, wu_ref[...], preferred_element_type=jnp.float32)

    def body_nog(ga_ref, gr_ref, ef_ref, b_ref, wu_ref, efo_ref):
        efn = ga_ref[...] - gr_ref[...] + ef_ref[...] + b_ref[...]
        efo_ref[...] = jnp.maximum(efn, 0.0)

    in_specs = [pl.BlockSpec((block_rows, d), lambda i: (i, 0)),
                pl.BlockSpec((block_rows, d), lambda i: ((i + hb) % grid, 0)),
                pl.BlockSpec((block_rows, d), lambda i: (i, 0)),
                pl.BlockSpec((1, d), lambda i: (0, 0)),
                pl.BlockSpec((d, d), lambda i: (0, 0))]
    if emit_g:
        return pl.pallas_call(
            body_g,
            grid=(grid,),
            in_specs=in_specs,
            out_specs=[pl.BlockSpec((block_rows, d), lambda i: (i, 0)),
                       pl.BlockSpec((block_rows, d), lambda i: (i, 0))],
            out_shape=[jax.ShapeDtypeStruct((e, d), jnp.float32),
                       jax.ShapeDtypeStruct((e, d), jnp.float32)],
        )(gathered, g, ef, b_upd, w_upd)
    return pl.pallas_call(
        body_nog,
        grid=(grid,),
        in_specs=in_specs,
        out_specs=pl.BlockSpec((block_rows, d), lambda i: (i, 0)),
        out_shape=jax.ShapeDtypeStruct((e, d), jnp.float32),
    )(gathered, g, ef, b_upd, w_upd)


def _tc_final(nf, msg, wfa, wfb, b_fin, block_rows):
    n, d = nf.shape

    def body(nf_ref, m_ref, wa_ref, wb_ref, b_ref, o_ref):
        acc = jnp.dot(nf_ref[...], wa_ref[...], preferred_element_type=jnp.float32)
        acc += jnp.dot(m_ref[...], wb_ref[...], preferred_element_type=jnp.float32)
        o_ref[...] = jnp.maximum(acc + b_ref[...], 0.0)

    return pl.pallas_call(
        body,
        grid=(n // block_rows,),
        in_specs=[pl.BlockSpec((block_rows, d), lambda i: (i, 0)),
                  pl.BlockSpec((block_rows, d), lambda i: (i, 0)),
                  pl.BlockSpec((d, d), lambda i: (0, 0)),
                  pl.BlockSpec((d, d), lambda i: (0, 0)),
                  pl.BlockSpec((1, d), lambda i: (0, 0))],
        out_specs=pl.BlockSpec((block_rows, d), lambda i: (i, 0)),
        out_shape=jax.ShapeDtypeStruct((n, d), jnp.float32),
    )(nf, msg, wfa, wfb, b_fin)


STEPS = 4
BLOCK_E = 640
BLOCK_N = 2000


def kernel(node_feature, edge_feature, W_init, b_init, W_upd, b_upd,
           W_fin, b_fin, edge_src, edge_dst):
    n, d = node_feature.shape

    w1a, w1b = W_init[:d], W_init[d:]
    wfa, wfb = W_fin[:d], W_fin[d:]
    b_init2 = b_init.reshape(1, -1)
    b_upd2 = b_upd.reshape(1, -1)
    b_fin2 = b_fin.reshape(1, -1)
    qzero = jnp.zeros((n, d), jnp.float32)

    p = _tc_matmul(node_feature, w1a, BLOCK_N)
    psrc = _sc_gather(p, edge_src, d)
    ef, g = _tc_init(psrc, edge_feature, w1b, b_init2, W_upd, BLOCK_E)

    for step in range(STEPS):
        gathered = _sc_step(g, edge_dst, edge_src, qzero)
        if step < STEPS - 1:
            ef, g = _tc_step(gathered, g, ef, b_upd2, W_upd, BLOCK_E, True)
        else:
            ef = _tc_step(gathered, g, ef, b_upd2, W_upd, BLOCK_E, False)

    msg = _sc_segsum(ef, edge_dst, qzero)
    return _tc_final(node_feature, msg, wfa, wfb, b_fin2, BLOCK_N)


# speedup vs baseline: 1.6136x; 1.6136x over previous
import functools

import jax
import jax.numpy as jnp
from jax import lax
from jax.experimental import pallas as pl
from jax.experimental.pallas import tpu as pltpu
from jax.experimental.pallas import tpu_sc as plsc

NC = 2
NS = 16
C = 128

_MESH = dict(core_axis_name="c", subcore_axis_name="s")


def _worker(cid, sid):
    return sid * NC + cid


def _sc_gather(table, idx, out_dim):
    e = idx.shape[0]
    n_chunks = e // C
    nloop = -(-n_chunks // (NC * NS))

    @functools.partial(
        pl.kernel,
        out_type=jax.ShapeDtypeStruct((e, out_dim), jnp.float32),
        mesh=plsc.VectorSubcoreMesh(**_MESH),
        scratch_types=[
            pltpu.VMEM((C,), jnp.int32),
            pltpu.VMEM((C, out_dim), jnp.float32),
            pltpu.SemaphoreType.DMA,
        ],
    )
    def k(table_hbm, idx_hbm, out_hbm, idx_v, rows_v, sem):
        wid = _worker(lax.axis_index("c"), lax.axis_index("s"))

        def body(i, carry):
            chunk = wid + i * (NC * NS)

            @pl.when(chunk < n_chunks)
            def _():
                base = chunk * C
                pltpu.sync_copy(idx_hbm.at[pl.ds(base, C)], idx_v)
                pltpu.async_copy(table_hbm.at[idx_v], rows_v, sem).wait()
                pltpu.sync_copy(rows_v, out_hbm.at[pl.ds(base, C)])

            return carry

        lax.fori_loop(0, nloop, body, 0)

    return k(table, idx)


def _sc_step(g, dst, src, qzero):
    e, d = g.shape
    n = qzero.shape[0]
    n_chunks = e // C
    scat_loop = -(-n_chunks // NS)
    half_chunks = (e // 2) // C
    gath_loop = -(-half_chunks // NS)
    ZR = 200
    z_chunks = n // ZR
    z_loop = -(-z_chunks // NS)

    @functools.partial(
        pl.kernel,
        out_type=jax.ShapeDtypeStruct((e, d), jnp.float32),
        mesh=plsc.VectorSubcoreMesh(**_MESH),
        scratch_types=[
            pltpu.VMEM((C,), jnp.int32),
            pltpu.VMEM((C, d), jnp.float32),
            pltpu.VMEM_SHARED((n, d), jnp.float32),
            pltpu.SemaphoreType.DMA,
        ],
    )
    def k(g_hbm, dst_hbm, src_hbm, qz_hbm, out_hbm, idx_v, buf_v, q_sh, sem):
        cid = lax.axis_index("c")
        sid = lax.axis_index("s")

        def zero(i, carry):
            chunk = sid + i * NS

            @pl.when(chunk < z_chunks)
            def _():
                base = chunk * ZR
                pltpu.sync_copy(qz_hbm.at[pl.ds(base, ZR)],
                                q_sh.at[pl.ds(base, ZR)])

            return carry

        lax.fori_loop(0, z_loop, zero, 0)
        plsc.subcore_barrier()

        def scat(i, carry):
            chunk = sid + i * NS

            @pl.when(chunk < n_chunks)
            def _():
                base = chunk * C
                pltpu.sync_copy(dst_hbm.at[pl.ds(base, C)], idx_v)
                pltpu.sync_copy(g_hbm.at[pl.ds(base, C)], buf_v)
                pltpu.sync_copy(buf_v, q_sh.at[idx_v], add=True)

            return carry

        lax.fori_loop(0, scat_loop, scat, 0)
        plsc.subcore_barrier()

        def gath(i, carry):
            chunk = sid + i * NS

            @pl.when(chunk < half_chunks)
            def _():
                base = cid * (e // 2) + chunk * C
                pltpu.sync_copy(src_hbm.at[pl.ds(base, C)], idx_v)
                pltpu.async_copy(q_sh.at[idx_v], buf_v, sem).wait()
                pltpu.sync_copy(buf_v, out_hbm.at[pl.ds(base, C)])

            return carry

        lax.fori_loop(0, gath_loop, gath, 0)

    return k(g, dst, src, qzero)


def _sc_segsum(rows, dst, qzero):
    e, d = rows.shape
    n = qzero.shape[0]
    n_chunks = e // C
    scat_loop = -(-n_chunks // NS)
    ZR = 200
    z_chunks = n // ZR
    z_loop = -(-z_chunks // NS)
    h_chunks = n // 2 // ZR
    h_loop = -(-h_chunks // NS)

    @functools.partial(
        pl.kernel,
        out_type=jax.ShapeDtypeStruct((n, d), jnp.float32),
        mesh=plsc.VectorSubcoreMesh(**_MESH),
        scratch_types=[
            pltpu.VMEM((C,), jnp.int32),
            pltpu.VMEM((C, d), jnp.float32),
            pltpu.VMEM_SHARED((n, d), jnp.float32),
        ],
    )
    def k(rows_hbm, dst_hbm, qz_hbm, q_hbm, idx_v, buf_v, q_sh):
        cid = lax.axis_index("c")
        sid = lax.axis_index("s")

        def zero(i, carry):
            chunk = sid + i * NS

            @pl.when(chunk < z_chunks)
            def _():
                base = chunk * ZR
                pltpu.sync_copy(qz_hbm.at[pl.ds(base, ZR)],
                                q_sh.at[pl.ds(base, ZR)])

            return carry

        lax.fori_loop(0, z_loop, zero, 0)
        plsc.subcore_barrier()

        def scat(i, carry):
            chunk = sid + i * NS

            @pl.when(chunk < n_chunks)
            def _():
                base = chunk * C
                pltpu.sync_copy(dst_hbm.at[pl.ds(base, C)], idx_v)
                pltpu.sync_copy(rows_hbm.at[pl.ds(base, C)], buf_v)
                pltpu.sync_copy(buf_v, q_sh.at[idx_v], add=True)

            return carry

        lax.fori_loop(0, scat_loop, scat, 0)
        plsc.subcore_barrier()

        def wb(i, carry):
            chunk = sid + i * NS

            @pl.when(chunk < h_chunks)
            def _():
                base = cid * (n // 2) + chunk * ZR
                pltpu.sync_copy(q_sh.at[pl.ds(base, ZR)],
                                q_hbm.at[pl.ds(base, ZR)])

            return carry

        lax.fori_loop(0, h_loop, wb, 0)

    return k(rows, dst, qzero)


def _tc_matmul(x, w, block_rows):
    m, kdim = x.shape
    _, nout = w.shape
    grid = m // block_rows

    def body(x_ref, w_ref, o_ref):
        o_ref[...] = jnp.dot(x_ref[...], w_ref[...],
                             preferred_element_type=jnp.float32)

    return pl.pallas_call(
        body,
        grid=(grid,),
        in_specs=[pl.BlockSpec((block_rows, kdim), lambda i: (i, 0)),
                  pl.BlockSpec((kdim, nout), lambda i: (0, 0))],
        out_specs=pl.BlockSpec((block_rows, nout), lambda i: (i, 0)),
        out_shape=jax.ShapeDtypeStruct((m, nout), jnp.float32),
    )(x, w)


def _tc_init(psrc, efeat, w1b, b_init, w_upd, block_rows):
    e, d = psrc.shape
    de = efeat.shape[1]
    grid = e // block_rows

    def body(p_ref, ef16_ref, w1b_ref, b_ref, wu_ref, ef_ref, g_ref):
        ef = p_ref[...] + jnp.dot(ef16_ref[...], w1b_ref[...],
                                  preferred_element_type=jnp.float32) + b_ref[...]
        ef = jnp.maximum(ef, 0.0)
        ef_ref[...] = ef
        g_ref[...] = jnp.dot(ef, wu_ref[...], preferred_element_type=jnp.float32)

    return pl.pallas_call(
        body,
        grid=(grid,),
        in_specs=[pl.BlockSpec((block_rows, d), lambda i: (i, 0)),
                  pl.BlockSpec((block_rows, de), lambda i: (i, 0)),
                  pl.BlockSpec((de, d), lambda i: (0, 0)),
                  pl.BlockSpec((1, d), lambda i: (0, 0)),
                  pl.BlockSpec((d, d), lambda i: (0, 0))],
        out_specs=[pl.BlockSpec((block_rows, d), lambda i: (i, 0)),
                   pl.BlockSpec((block_rows, d), lambda i: (i, 0))],
        out_shape=[jax.ShapeDtypeStruct((e, d), jnp.float32),
                   jax.ShapeDtypeStruct((e, d), jnp.float32)],
    )(psrc, efeat, w1b, b_init, w_upd)


def _tc_step(gathered, g, ef, b_upd, w_upd, block_rows, emit_g):
    e, d = g.shape
    grid = e // block_rows
    hb = grid // 2

    def body_g(ga_ref, gr_ref, ef_ref, b_ref, wu_ref, efo_ref, go_ref):
        efn = ga_ref[...] - gr_ref[...] + ef_ref[...] + b_ref[...]
        efn = jnp.maximum(efn, 0.0)
        efo_ref[...] = efn
        go_ref[...] = jnp.dot(efn, wu_ref[...], preferred_element_type=jnp.float32)

    def body_nog(ga_ref, gr_ref, ef_ref, b_ref, wu_ref, efo_ref):
        efn = ga_ref[...] - gr_ref[...] + ef_ref[...] + b_ref[...]
        efo_ref[...] = jnp.maximum(efn, 0.0)

    in_specs = [pl.BlockSpec((block_rows, d), lambda i: (i, 0)),
                pl.BlockSpec((block_rows, d), lambda i: ((i + hb) % grid, 0)),
                pl.BlockSpec((block_rows, d), lambda i: (i, 0)),
                pl.BlockSpec((1, d), lambda i: (0, 0)),
                pl.BlockSpec((d, d), lambda i: (0, 0))]
    if emit_g:
        return pl.pallas_call(
            body_g,
            grid=(grid,),
            in_specs=in_specs,
            out_specs=[pl.BlockSpec((block_rows, d), lambda i: (i, 0)),
                       pl.BlockSpec((block_rows, d), lambda i: (i, 0))],
            out_shape=[jax.ShapeDtypeStruct((e, d), jnp.float32),
                       jax.ShapeDtypeStruct((e, d), jnp.float32)],
        )(gathered, g, ef, b_upd, w_upd)
    return pl.pallas_call(
        body_nog,
        grid=(grid,),
        in_specs=in_specs,
        out_specs=pl.BlockSpec((block_rows, d), lambda i: (i, 0)),
        out_shape=jax.ShapeDtypeStruct((e, d), jnp.float32),
    )(gathered, g, ef, b_upd, w_upd)


def _tc_final(nf, msg, wfa, wfb, b_fin, block_rows):
    n, d = nf.shape

    def body(nf_ref, m_ref, wa_ref, wb_ref, b_ref, o_ref):
        acc = jnp.dot(nf_ref[...], wa_ref[...], preferred_element_type=jnp.float32)
        acc += jnp.dot(m_ref[...], wb_ref[...], preferred_element_type=jnp.float32)
        o_ref[...] = jnp.maximum(acc + b_ref[...], 0.0)

    return pl.pallas_call(
        body,
        grid=(n // block_rows,),
        in_specs=[pl.BlockSpec((block_rows, d), lambda i: (i, 0)),
                  pl.BlockSpec((block_rows, d), lambda i: (i, 0)),
                  pl.BlockSpec((d, d), lambda i: (0, 0)),
                  pl.BlockSpec((d, d), lambda i: (0, 0)),
                  pl.BlockSpec((1, d), lambda i: (0, 0))],
        out_specs=pl.BlockSpec((block_rows, d), lambda i: (i, 0)),
        out_shape=jax.ShapeDtypeStruct((n, d), jnp.float32),
    )(nf, msg, wfa, wfb, b_fin)


STEPS = 4
BLOCK_E = 640
BLOCK_N = 2000


def kernel(node_feature, edge_feature, W_init, b_init, W_upd, b_upd,
           W_fin, b_fin, edge_src, edge_dst):
    n, d = node_feature.shape

    w1a, w1b = W_init[:d], W_init[d:]
    wfa, wfb = W_fin[:d], W_fin[d:]
    b_init2 = b_init.reshape(1, -1)
    b_upd2 = b_upd.reshape(1, -1)
    b_fin2 = b_fin.reshape(1, -1)
    qzero = jnp.zeros((n, d), jnp.float32)

    p = _tc_matmul(node_feature, w1a, BLOCK_N)
    psrc = _sc_gather(p, edge_src, d)
    ef, g = _tc_init(psrc, edge_feature, w1b, b_init2, W_upd, BLOCK_E)

    for step in range(STEPS):
        gathered = _sc_step(g, edge_dst, edge_src, qzero)
        if step < STEPS - 1:
            ef, g = _tc_step(gathered, g, ef, b_upd2, W_upd, BLOCK_E, True)
        else:
            ef = _tc_step(gathered, g, ef, b_upd2, W_upd, BLOCK_E, False)

    msg = _sc_segsum(ef, edge_dst, qzero)
    return _tc_final(node_feature, msg, wfa, wfb, b_fin2, BLOCK_N)


# speedup vs baseline: 2.6983x; 1.6722x over previous
import functools

import jax
import jax.numpy as jnp
from jax import lax
from jax.experimental import pallas as pl
from jax.experimental.pallas import tpu as pltpu
from jax.experimental.pallas import tpu_sc as plsc

NC = 2
NS = 16
C = 128
K = 3
ZR = 200


def _sc_scatter_split(rows, dst, qzero):
    e, d = rows.shape
    n = qzero.shape[0]
    eh = e // 2
    n_chunks = eh // C
    nloop = -(-n_chunks // NS)
    rounds = -(-nloop // K)
    z_chunks = n // ZR
    z_loop = -(-z_chunks // NS)

    scratch = ([pltpu.VMEM((C,), jnp.int32) for _ in range(K)]
               + [pltpu.VMEM((C, d), jnp.float32) for _ in range(K)]
               + [pltpu.VMEM_SHARED((n, d), jnp.float32)]
               + [pltpu.SemaphoreType.DMA for _ in range(2 * K)])

    @functools.partial(
        pl.kernel,
        out_type=jax.ShapeDtypeStruct((NC, n, d), jnp.float32),
        mesh=plsc.VectorSubcoreMesh(core_axis_name="c", subcore_axis_name="s"),
        scratch_types=scratch,
    )
    def k(rows_hbm, dst_hbm, qz_hbm, out_hbm, *sc):
        idx_v = sc[0:K]
        buf_v = sc[K:2 * K]
        q_sh = sc[2 * K]
        sem_i = sc[2 * K + 1:2 * K + 1 + K]
        sem_r = sc[2 * K + 1 + K:2 * K + 1 + 2 * K]
        cid = lax.axis_index("c")
        sid = lax.axis_index("s")
        ebase = cid * eh

        def zero(i, carry):
            chunk = sid + i * NS

            @pl.when(chunk < z_chunks)
            def _():
                pltpu.sync_copy(qz_hbm.at[pl.ds(chunk * ZR, ZR)],
                                q_sh.at[pl.ds(chunk * ZR, ZR)])

            return carry

        lax.fori_loop(0, z_loop, zero, 0)
        plsc.subcore_barrier()

        def start_in(b, j):
            @pl.when(sid + j * NS < n_chunks)
            def _():
                base = ebase + (sid + j * NS) * C
                pltpu.async_copy(dst_hbm.at[pl.ds(base, C)], idx_v[b], sem_i[b])
                pltpu.async_copy(rows_hbm.at[pl.ds(base, C)], buf_v[b], sem_r[b])

        for b in range(K):
            start_in(b, b)

        def rnd(r, carry):
            for b in range(K):
                j = r * K + b

                @pl.when(sid + j * NS < n_chunks)
                def _():
                    base = ebase + (sid + j * NS) * C
                    pltpu.make_async_copy(dst_hbm.at[pl.ds(base, C)],
                                          idx_v[b], sem_i[b]).wait()
                    pltpu.make_async_copy(rows_hbm.at[pl.ds(base, C)],
                                          buf_v[b], sem_r[b]).wait()
                    pltpu.sync_copy(buf_v[b], q_sh.at[idx_v[b]], add=True)
                    start_in(b, j + K)

            return carry

        lax.fori_loop(0, rounds, rnd, 0)
        plsc.subcore_barrier()

        def wb(i, carry):
            chunk = sid + i * NS

            @pl.when(chunk < z_chunks)
            def _():
                pltpu.sync_copy(q_sh.at[pl.ds(chunk * ZR, ZR)],
                                out_hbm.at[cid, pl.ds(chunk * ZR, ZR)])

            return carry

        lax.fori_loop(0, z_loop, wb, 0)

    return k(rows, dst, qzero)


def _sc_gather_update(table, idx, h):
    e, d = h.shape
    eh = e // 2
    n_chunks = eh // C
    nloop = -(-n_chunks // NS)
    rounds = -(-nloop // K)

    scratch = ([pltpu.VMEM((C,), jnp.int32) for _ in range(K)]
               + [pltpu.VMEM((C, d), jnp.float32) for _ in range(2 * K)]
               + [pltpu.SemaphoreType.DMA for _ in range(4 * K)])

    @functools.partial(
        pl.kernel,
        out_type=jax.ShapeDtypeStruct((e, d), jnp.float32),
        mesh=plsc.VectorSubcoreMesh(core_axis_name="c", subcore_axis_name="s"),
        scratch_types=scratch,
    )
    def k(table_hbm, idx_hbm, h_hbm, out_hbm, *sc):
        idx_v = sc[0:K]
        gbuf = sc[K:2 * K]
        hbuf = sc[2 * K:3 * K]
        sem_i = sc[3 * K:4 * K]
        sem_g = sc[4 * K:5 * K]
        sem_h = sc[5 * K:6 * K]
        sem_o = sc[6 * K:7 * K]
        cid = lax.axis_index("c")
        sid = lax.axis_index("s")
        ebase = cid * eh

        def start_in(b, j):
            @pl.when(sid + j * NS < n_chunks)
            def _():
                base = ebase + (sid + j * NS) * C
                pltpu.async_copy(idx_hbm.at[pl.ds(base, C)], idx_v[b], sem_i[b])
                pltpu.async_copy(h_hbm.at[pl.ds(base, C)], hbuf[b], sem_h[b])

        for b in range(K):
            start_in(b, b)

        def rnd(r, carry):
            for b in range(K):
                j = r * K + b

                @pl.when(sid + j * NS < n_chunks)
                def _():
                    base = ebase + (sid + j * NS) * C
                    pltpu.make_async_copy(idx_hbm.at[pl.ds(base, C)],
                                          idx_v[b], sem_i[b]).wait()
                    pltpu.async_copy(table_hbm.at[idx_v[b]], gbuf[b], sem_g[b])
                    pltpu.make_async_copy(h_hbm.at[pl.ds(base, C)],
                                          hbuf[b], sem_h[b]).wait()
                    pltpu.make_async_copy(table_hbm.at[idx_v[b]],
                                          gbuf[b], sem_g[b]).wait()

                    def upd(rr, carry2):
                        for c8 in range(d // 16):
                            sl = pl.ds(c8 * 16, 16)
                            gbuf[b][rr, sl] = jnp.maximum(
                                gbuf[b][rr, sl] + hbuf[b][rr, sl], 0.0)
                        return carry2

                    lax.fori_loop(0, C, upd, 0)
                    pltpu.async_copy(gbuf[b], out_hbm.at[pl.ds(base, C)],
                                     sem_o[b])
                    pltpu.make_async_copy(gbuf[b], out_hbm.at[pl.ds(base, C)],
                                          sem_o[b]).wait()
                    start_in(b, j + K)

            return carry

        lax.fori_loop(0, rounds, rnd, 0)

    return k(table, idx, h)


def _tc_matmul(x, w, block_rows):
    m, kdim = x.shape
    _, nout = w.shape

    def body(x_ref, w_ref, o_ref):
        o_ref[...] = jnp.dot(x_ref[...], w_ref[...],
                             preferred_element_type=jnp.float32)

    return pl.pallas_call(
        body,
        grid=(m // block_rows,),
        in_specs=[pl.BlockSpec((block_rows, kdim), lambda i: (i, 0)),
                  pl.BlockSpec((kdim, nout), lambda i: (0, 0))],
        out_specs=pl.BlockSpec((block_rows, nout), lambda i: (i, 0)),
        out_shape=jax.ShapeDtypeStruct((m, nout), jnp.float32),
    )(x, w)


def _tc_h0(efeat, w1b, b_init, block_rows):
    e, de = efeat.shape
    d = w1b.shape[1]

    def body(ef_ref, w_ref, b_ref, o_ref):
        o_ref[...] = jnp.dot(ef_ref[...], w_ref[...],
                             preferred_element_type=jnp.float32) + b_ref[...]

    return pl.pallas_call(
        body,
        grid=(e // block_rows,),
        in_specs=[pl.BlockSpec((block_rows, de), lambda i: (i, 0)),
                  pl.BlockSpec((de, d), lambda i: (0, 0)),
                  pl.BlockSpec((1, d), lambda i: (0, 0))],
        out_specs=pl.BlockSpec((block_rows, d), lambda i: (i, 0)),
        out_shape=jax.ShapeDtypeStruct((e, d), jnp.float32),
    )(efeat, w1b, b_init)


def _tc_h(ef, w_upd, b_upd, block_rows):
    e, d = ef.shape
    grid = e // block_rows
    hb = grid // 2

    def body(efa_ref, efb_ref, b_ref, w_ref, o_ref):
        efa = efa_ref[...]
        efb = efb_ref[...]
        o_ref[0, ...] = efa + b_ref[...] - jnp.dot(
            efb, w_ref[...], preferred_element_type=jnp.float32)
        o_ref[1, ...] = efb + b_ref[...] - jnp.dot(
            efa, w_ref[...], preferred_element_type=jnp.float32)

    out = pl.pallas_call(
        body,
        grid=(hb,),
        in_specs=[pl.BlockSpec((block_rows, d), lambda i: (i, 0)),
                  pl.BlockSpec((block_rows, d), lambda i: (i + hb, 0)),
                  pl.BlockSpec((1, d), lambda i: (0, 0)),
                  pl.BlockSpec((d, d), lambda i: (0, 0))],
        out_specs=pl.BlockSpec((2, block_rows, d), lambda i: (0, i, 0)),
        out_shape=jax.ShapeDtypeStruct((2, e // 2, d), jnp.float32),
    )(ef, ef, b_upd, w_upd)
    return out.reshape(e, d)


def _tc_qcomb(a0, a1, w_upd, block_rows):
    n, d = a0.shape

    def body(a0_ref, a1_ref, w_ref, o_ref):
        o_ref[...] = jnp.dot(a0_ref[...] + a1_ref[...], w_ref[...],
                             preferred_element_type=jnp.float32)

    return pl.pallas_call(
        body,
        grid=(n // block_rows,),
        in_specs=[pl.BlockSpec((block_rows, d), lambda i: (i, 0)),
                  pl.BlockSpec((block_rows, d), lambda i: (i, 0)),
                  pl.BlockSpec((d, d), lambda i: (0, 0))],
        out_specs=pl.BlockSpec((block_rows, d), lambda i: (i, 0)),
        out_shape=jax.ShapeDtypeStruct((n, d), jnp.float32),
    )(a0, a1, w_upd)


def _tc_final(nf, m0, m1, wfa, wfb, b_fin, block_rows):
    n, d = nf.shape

    def body(nf_ref, m0_ref, m1_ref, wa_ref, wb_ref, b_ref, o_ref):
        acc = jnp.dot(nf_ref[...], wa_ref[...], preferred_element_type=jnp.float32)
        acc += jnp.dot(m0_ref[...] + m1_ref[...], wb_ref[...],
                       preferred_element_type=jnp.float32)
        o_ref[...] = jnp.maximum(acc + b_ref[...], 0.0)

    return pl.pallas_call(
        body,
        grid=(n // block_rows,),
        in_specs=[pl.BlockSpec((block_rows, d), lambda i: (i, 0)),
                  pl.BlockSpec((block_rows, d), lambda i: (i, 0)),
                  pl.BlockSpec((block_rows, d), lambda i: (i, 0)),
                  pl.BlockSpec((d, d), lambda i: (0, 0)),
                  pl.BlockSpec((d, d), lambda i: (0, 0)),
                  pl.BlockSpec((1, d), lambda i: (0, 0))],
        out_specs=pl.BlockSpec((block_rows, d), lambda i: (i, 0)),
        out_shape=jax.ShapeDtypeStruct((n, d), jnp.float32),
    )(nf, m0, m1, wfa, wfb, b_fin)


STEPS = 4
BLOCK_E = 640
BLOCK_N = 2000


def kernel(node_feature, edge_feature, W_init, b_init, W_upd, b_upd,
           W_fin, b_fin, edge_src, edge_dst):
    n, d = node_feature.shape

    w1a, w1b = W_init[:d], W_init[d:]
    wfa, wfb = W_fin[:d], W_fin[d:]
    b_init2 = b_init.reshape(1, -1)
    b_upd2 = b_upd.reshape(1, -1)
    b_fin2 = b_fin.reshape(1, -1)
    qzero = jnp.zeros((n, d), jnp.float32)

    p = _tc_matmul(node_feature, w1a, BLOCK_N)
    h0 = _tc_h0(edge_feature, w1b, b_init2, BLOCK_E)
    ef = _sc_gather_update(p, edge_src, h0)

    for _ in range(STEPS):
        parts = _sc_scatter_split(ef, edge_dst, qzero)
        hh = _tc_h(ef, W_upd, b_upd2, BLOCK_E)
        q = _tc_qcomb(parts[0], parts[1], W_upd, BLOCK_N)
        ef = _sc_gather_update(q, edge_src, hh)

    parts = _sc_scatter_split(ef, edge_dst, qzero)
    return _tc_final(node_feature, parts[0], parts[1], wfa, wfb, b_fin2, BLOCK_N)
